# baseline (speedup 1.0000x reference)
import functools

import jax
import jax.numpy as jnp
from jax import lax
from jax.experimental import pallas as pl
from jax.experimental.pallas import tpu as pltpu
from jax.experimental.pallas import tpu_sc as plsc

_L = 16
_RBLK = 128
_NBUF = 2


def _balance_sc(y, aux):
    B, F = y.shape
    info = plsc.get_sparse_core_info()
    nc, ns = info.num_cores, info.num_subcores
    nw = nc * ns
    rows_pw = B // nw
    nblk = rows_pw // _RBLK
    nch = F // _L

    mesh = plsc.VectorSubcoreMesh(core_axis_name="c", subcore_axis_name="s")

    @functools.partial(
        pl.kernel,
        mesh=mesh,
        out_type=jax.ShapeDtypeStruct((B, F), jnp.float32),
        scratch_types=(
            [pltpu.VMEM((_RBLK, F), jnp.float32) for _ in range(2 * _NBUF)]
            + [pltpu.VMEM((12, _L), jnp.float32)]
            + [pltpu.SemaphoreType.DMA for _ in range(2 * _NBUF)]
        ),
    )
    def run(y_hbm, aux_hbm, out_hbm, in0, in1, ob0, ob1, aux_v, si0, si1, so0, so1):
        inb = (in0, in1)
        outb = (ob0, ob1)
        sin = (si0, si1)
        sout = (so0, so1)
        wid = lax.axis_index("s") * nc + lax.axis_index("c")
        base = wid * rows_pw

        pltpu.sync_copy(aux_hbm, aux_v)
        w = [aux_v[k, :] for k in range(nch)]
        cv = aux_v[4, :]
        oneh = [aux_v[5 + k, :] for k in range(nch)]
        ii = lax.iota(jnp.int32, _L)
        bfly = [jnp.bitwise_xor(ii, 1 << t) for t in range(4)]

        def copy_in(g):
            return pltpu.make_async_copy(
                y_hbm.at[pl.ds(base + g * _RBLK, _RBLK)], inb[g % _NBUF], sin[g % _NBUF]
            )

        def copy_out(g):
            return pltpu.make_async_copy(
                outb[g % _NBUF], out_hbm.at[pl.ds(base + g * _RBLK, _RBLK)], sout[g % _NBUF]
            )

        for b in range(min(_NBUF, nblk)):
            copy_in(b).start()

        for g in range(nblk):
            s = g % _NBUF
            copy_in(g).wait()
            if g >= _NBUF:
                copy_out(g - _NBUF).wait()
            src, dst = inb[s], outb[s]

            def row(r, carry):
                ys = [src[r, pl.ds(k * _L, _L)] for k in range(nch)]
                p = ys[0] * w[0] + cv
                for k in range(1, nch):
                    p = p + ys[k] * w[k]
                dnums = lax.GatherDimensionNumbers(
                    offset_dims=(), collapsed_slice_dims=(0,), start_index_map=(0,)
                )
                for m in bfly:
                    p = p + lax.gather(
                        p, m[:, None], dnums, (1,),
                        unique_indices=True, indices_are_sorted=False,
                        mode=lax.GatherScatterMode.PROMISE_IN_BOUNDS,
                    )
                for k in range(nch):
                    dst[r, pl.ds(k * _L, _L)] = ys[k] + p * oneh[k]
                return carry

            lax.fori_loop(0, _RBLK, row, 0)

            copy_out(g).start()
            if g + _NBUF < nblk:
                copy_in(g + _NBUF).start()

        for g in range(max(nblk - _NBUF, 0), nblk):
            copy_out(g).wait()

    return run(y, aux)


def kernel(y, means, stds, asset_idx, liability_idx, equity_idx, slack_idx):
    f32 = jnp.float32
    B, F = y.shape
    sign = (
        jnp.zeros((F,), f32)
        .at[asset_idx].set(1.0)
        .at[liability_idx].set(-1.0)
        .at[equity_idx].set(-1.0)
    )
    inv = 1.0 / stds[slack_idx]
    w = sign * stds * inv
    c = jnp.sum(sign * means) * inv
    oneh = (jnp.arange(F) == slack_idx).astype(f32)
    aux = jnp.zeros((12, _L), f32)
    aux = aux.at[0:4].set(w.reshape(4, _L))
    aux = aux.at[4, 0].set(c)
    aux = aux.at[5:9].set(oneh.reshape(4, _L))
    return _balance_sc(y.astype(f32), aux)
